# baseline (speedup 1.0000x reference)
import functools

import jax
import jax.numpy as jnp
from jax import lax
from jax.experimental import pallas as pl
from jax.experimental.pallas import tpu as pltpu
from jax.experimental.pallas import tpu_sc as plsc

VOCAB = 100000
EMBED = 64
EPS = 1e-12

NC = 2
NS = 16
NW = NC * NS
LANES = 16

SEQ = 16384
WIDTH = 50
B = SEQ * WIDTH
BPW = B // NW
SPW = SEQ // NW
CHUNK = 100
CSEQ = CHUNK // WIDTH
NCHUNK = BPW // CHUNK
SUB = 1
SUBN = CHUNK // SUB
NBUF = 4


def _rsqrt(x):
    i = lax.bitcast_convert_type(x, jnp.int32)
    i = jnp.int32(0x5F3759DF) - lax.shift_right_logical(i, 1)
    y = lax.bitcast_convert_type(i, jnp.float32)
    for _ in range(3):
        y = y * (1.5 - 0.5 * x * y * y)
    return y


_UNROLL = 8


def _emb_ln_body(ids_hbm, table_hbm, scale_hbm, bias_hbm, out_hbm, *rest):
    idxb = rest[0:NBUF]
    rowsb = rest[NBUF:2 * NBUF]
    outb = rest[2 * NBUF:3 * NBUF]
    sbv = rest[3 * NBUF]
    isem = rest[3 * NBUF + 1:4 * NBUF + 1]
    gsem = rest[4 * NBUF + 1:5 * NBUF + 1]
    osem = rest[5 * NBUF + 1:6 * NBUF + 1]

    wid = lax.axis_index("s") * NC + lax.axis_index("c")
    base_seq = wid * SPW
    base_idxrow = wid * (BPW // SUBN)

    pltpu.sync_copy(scale_hbm, sbv.at[pl.ds(0, EMBED)])
    pltpu.sync_copy(bias_hbm, sbv.at[pl.ds(EMBED, EMBED)])

    def start_idx(c, b):
        pltpu.async_copy(ids_hbm.at[pl.ds(base_idxrow + c * SUB, SUB)],
                         idxb[b], isem[b])

    def wait_idx(b):
        pltpu.make_async_copy(ids_hbm.at[pl.ds(base_idxrow, SUB)],
                              idxb[b], isem[b]).wait()

    def fire_gather(b):
        for k in range(SUB):
            pltpu.async_copy(table_hbm.at[idxb[b].at[k]],
                             rowsb[b].at[pl.ds(k * SUBN, SUBN)], gsem[b])

    def wait_gather(b):
        for k in range(SUB):
            pltpu.make_async_copy(table_hbm.at[idxb[b].at[k]],
                                  rowsb[b].at[pl.ds(k * SUBN, SUBN)],
                                  gsem[b]).wait()

    def start_out(c, b):
        pltpu.async_copy(outb[b],
                         out_hbm.at[pl.ds(base_seq + c * CSEQ, CSEQ)],
                         osem[b])

    def wait_out(b):
        pltpu.make_async_copy(outb[b],
                              out_hbm.at[pl.ds(base_seq, CSEQ)],
                              osem[b]).wait()

    def compute_chunk(b):
        rows = rowsb[b]
        ob = outb[b]
        sv = [sbv[pl.ds(LANES * k, LANES)] for k in range(4)]
        bv = [sbv[pl.ds(EMBED + LANES * k, LANES)] for k in range(4)]

        @plsc.parallel_loop(0, CHUNK, 1, unroll=_UNROLL)
        def row_body(r):
            v = [rows[r, pl.ds(LANES * k, LANES)] for k in range(4)]
            s = (v[0] + v[1]) + (v[2] + v[3])
            mean = jnp.full((LANES,), jnp.sum(s) * (1.0 / EMBED))
            d = [v[k] - mean for k in range(4)]
            q = (d[0] * d[0] + d[1] * d[1]) + (d[2] * d[2] + d[3] * d[3])
            var = jnp.full((LANES,), jnp.sum(q) * (1.0 / EMBED))
            rstd = _rsqrt(var + EPS)
            rs = r // WIDTH
            rw = r - rs * WIDTH
            for k in range(4):
                ob[rs, pl.ds(rw * EMBED + LANES * k, LANES)] = \
                    d[k] * rstd * sv[k] + bv[k]

    for b in range(NBUF):
        start_idx(b, b)
        wait_idx(b)
        fire_gather(b)

    def chunk_iter(i, carry):
        for b in range(NBUF):
            cc = i * NBUF + b
            wait_gather(b)

            @pl.when(cc + NBUF < NCHUNK)
            def _():
                start_idx(cc + NBUF, b)

            compute_chunk(b)

            @pl.when(cc >= NBUF)
            def _():
                wait_out(b)

            start_out(cc, b)

            @pl.when(cc + NBUF < NCHUNK)
            def _():
                wait_idx(b)
                fire_gather(b)

        return carry

    lax.fori_loop(0, NCHUNK // NBUF, chunk_iter, 0)

    for b in range(NBUF):
        wait_out(b)


_emb_ln = functools.partial(
    pl.kernel,
    mesh=plsc.VectorSubcoreMesh(core_axis_name="c", subcore_axis_name="s"),
    compiler_params=pltpu.CompilerParams(needs_layout_passes=False,
                                         use_tc_tiling_on_sc=False),
    out_type=jax.ShapeDtypeStruct((SEQ, WIDTH * EMBED), jnp.float32),
    scratch_types=(
        [pltpu.VMEM((SUB, SUBN), jnp.int32) for _ in range(NBUF)]
        + [pltpu.VMEM((CHUNK, EMBED), jnp.float32) for _ in range(NBUF)]
        + [pltpu.VMEM((CSEQ, WIDTH * EMBED), jnp.float32)
           for _ in range(NBUF)]
        + [pltpu.VMEM((2 * EMBED,), jnp.float32)]
        + [pltpu.SemaphoreType.DMA for _ in range(3 * NBUF)]
    ),
)(_emb_ln_body)


def kernel(input_ids, attention_mask, table, ln_scale, ln_bias):
    del attention_mask
    seq, width = input_ids.shape
    assert seq * width == B and table.shape == (VOCAB, EMBED)
    ids = input_ids.astype(jnp.int32).reshape(B // SUBN, SUBN)
    out = _emb_ln(ids, table, ln_scale, ln_bias)
    return out.reshape(seq, width, EMBED)


__all__ = ["kernel"]


# speedup vs baseline: 1.3101x; 1.3101x over previous
import functools

import jax
import jax.numpy as jnp
from jax import lax
from jax.experimental import pallas as pl
from jax.experimental.pallas import tpu as pltpu
from jax.experimental.pallas import tpu_sc as plsc

VOCAB = 100000
EMBED = 64
EPS = 1e-12

NC = 2
NS = 16
NW = NC * NS
LANES = 16

SEQ = 16384
WIDTH = 50
B = SEQ * WIDTH
BPW = B // NW
SPW = SEQ // NW
CHUNK = 200
CSEQ = CHUNK // WIDTH
NCHUNK = BPW // CHUNK
SUB = 2
SUBN = CHUNK // SUB
NBUF = 2


def _rsqrt(x):
    i = lax.bitcast_convert_type(x, jnp.int32)
    i = jnp.int32(0x5F3759DF) - lax.shift_right_logical(i, 1)
    y = lax.bitcast_convert_type(i, jnp.float32)
    for _ in range(3):
        y = y * (1.5 - 0.5 * x * y * y)
    return y


_UNROLL = 8


def _emb_ln_body(ids_hbm, table_hbm, scale_hbm, bias_hbm, out_hbm,
                 idx0, idx1, rows0, rows1, ob0, ob1, sbv,
                 isem0, isem1, gsem0, gsem1, osem0, osem1):
    idxb = (idx0, idx1)
    rowsb = (rows0, rows1)
    outb = (ob0, ob1)
    isem = (isem0, isem1)
    gsem = (gsem0, gsem1)
    osem = (osem0, osem1)

    wid = lax.axis_index("s") * NC + lax.axis_index("c")
    base_seq = wid * SPW
    base_idxrow = wid * (BPW // SUBN)

    pltpu.sync_copy(scale_hbm, sbv.at[pl.ds(0, EMBED)])
    pltpu.sync_copy(bias_hbm, sbv.at[pl.ds(EMBED, EMBED)])

    def start_idx(c, b):
        pltpu.async_copy(ids_hbm.at[pl.ds(base_idxrow + c * SUB, SUB)],
                         idxb[b], isem[b])

    def wait_idx(b):
        pltpu.make_async_copy(ids_hbm.at[pl.ds(base_idxrow, SUB)],
                              idxb[b], isem[b]).wait()

    def fire_gather(b):
        for k in range(SUB):
            pltpu.async_copy(table_hbm.at[idxb[b].at[k]],
                             rowsb[b].at[pl.ds(k * SUBN, SUBN)], gsem[b])

    def wait_gather(b):
        for k in range(SUB):
            pltpu.make_async_copy(table_hbm.at[idxb[b].at[k]],
                                  rowsb[b].at[pl.ds(k * SUBN, SUBN)],
                                  gsem[b]).wait()

    def start_out(c, b):
        pltpu.async_copy(outb[b],
                         out_hbm.at[pl.ds(base_seq + c * CSEQ, CSEQ)],
                         osem[b])

    def wait_out(b):
        pltpu.make_async_copy(outb[b],
                              out_hbm.at[pl.ds(base_seq, CSEQ)],
                              osem[b]).wait()

    def compute_chunk(b):
        rows = rowsb[b]
        ob = outb[b]
        sv = [sbv[pl.ds(LANES * k, LANES)] for k in range(4)]
        bv = [sbv[pl.ds(EMBED + LANES * k, LANES)] for k in range(4)]

        @plsc.parallel_loop(0, CHUNK, 1, unroll=_UNROLL)
        def row_body(r):
            v = [rows[r, pl.ds(LANES * k, LANES)] for k in range(4)]
            s = (v[0] + v[1]) + (v[2] + v[3])
            mean = jnp.full((LANES,), jnp.sum(s) * (1.0 / EMBED))
            d = [v[k] - mean for k in range(4)]
            q = (d[0] * d[0] + d[1] * d[1]) + (d[2] * d[2] + d[3] * d[3])
            var = jnp.full((LANES,), jnp.sum(q) * (1.0 / EMBED))
            rstd = _rsqrt(var + EPS)
            rs = r // WIDTH
            rw = r - rs * WIDTH
            for k in range(4):
                ob[rs, pl.ds(rw * EMBED + LANES * k, LANES)] = \
                    d[k] * rstd * sv[k] + bv[k]

    for b in range(NBUF):
        start_idx(b, b)
        wait_idx(b)
        fire_gather(b)

    def chunk_iter(i, carry):
        for b in range(NBUF):
            cc = i * NBUF + b
            wait_gather(b)

            @pl.when(cc + NBUF < NCHUNK)
            def _():
                start_idx(cc + NBUF, b)

            compute_chunk(b)

            @pl.when(cc >= NBUF)
            def _():
                wait_out(b)

            start_out(cc, b)

            @pl.when(cc + NBUF < NCHUNK)
            def _():
                wait_idx(b)
                fire_gather(b)

        return carry

    lax.fori_loop(0, NCHUNK // NBUF, chunk_iter, 0)

    for b in range(NBUF):
        wait_out(b)


_emb_ln = functools.partial(
    pl.kernel,
    mesh=plsc.VectorSubcoreMesh(core_axis_name="c", subcore_axis_name="s"),
    compiler_params=pltpu.CompilerParams(needs_layout_passes=False,
                                         use_tc_tiling_on_sc=False),
    out_type=jax.ShapeDtypeStruct((SEQ, WIDTH * EMBED), jnp.float32),
    scratch_types=[
        pltpu.VMEM((SUB, SUBN), jnp.int32),
        pltpu.VMEM((SUB, SUBN), jnp.int32),
        pltpu.VMEM((CHUNK, EMBED), jnp.float32),
        pltpu.VMEM((CHUNK, EMBED), jnp.float32),
        pltpu.VMEM((CSEQ, WIDTH * EMBED), jnp.float32),
        pltpu.VMEM((CSEQ, WIDTH * EMBED), jnp.float32),
        pltpu.VMEM((2 * EMBED,), jnp.float32),
        pltpu.SemaphoreType.DMA,
        pltpu.SemaphoreType.DMA,
        pltpu.SemaphoreType.DMA,
        pltpu.SemaphoreType.DMA,
        pltpu.SemaphoreType.DMA,
        pltpu.SemaphoreType.DMA,
    ],
)(_emb_ln_body)


def kernel(input_ids, attention_mask, table, ln_scale, ln_bias):
    del attention_mask
    seq, width = input_ids.shape
    assert seq * width == B and table.shape == (VOCAB, EMBED)
    ids = input_ids.astype(jnp.int32).reshape(B // SUBN, SUBN)
    out = _emb_ln(ids, table, ln_scale, ln_bias)
    return out.reshape(seq, width, EMBED)


__all__ = ["kernel"]
